# Initial kernel scaffold; baseline (speedup 1.0000x reference)
import jax
import jax.numpy as jnp
from jax.experimental import pallas as pl


def kernel(user_id, item_id, gender, age_range, city, cluster_id, user_numeric, item_text_feat, user_sequence, sequence_mask, user_table, gender_table, age_table, city_table, cluster_table, item_table, position_table, W_num, b_num, g_num, be_num, W_text, b_text, g_text, be_text, W_seq, b_seq, W_a1, b_a1, W_a2, b_a2, W_u1, b_u1, W_u2, b_u2, W_i1, b_i1, W_i2, b_i2):
    raise NotImplementedError("write your pallas kernel here")


# speedup vs baseline: 1.6370x; 1.6370x over previous
import functools

import jax
import jax.numpy as jnp
from jax import lax
from jax.experimental import pallas as pl
from jax.experimental.pallas import tpu as pltpu
from jax.experimental.pallas import tpu_sc as plsc

B = 4096
D = 64
SEQ = 50
NC = 2
NS = 16
NW = NC * NS
BPW = B // NW
SROWS = SEQ * B
SPW = SROWS // NW
SCHUNK = 640
NCHUNK = SPW // SCHUNK

BB = 256
NB = B // BB


def _sc_gather(user_table, item_table, uidx, iidx, sidx3):
    mesh = plsc.VectorSubcoreMesh(core_axis_name="c", subcore_axis_name="s")

    @functools.partial(
        pl.kernel,
        mesh=mesh,
        out_type=(
            jax.ShapeDtypeStruct((B, D), jnp.float32),
            jax.ShapeDtypeStruct((B, D), jnp.float32),
            jax.ShapeDtypeStruct((SROWS, D), jnp.float32),
        ),
        scratch_types=[
            pltpu.VMEM((BPW,), jnp.int32),
            pltpu.VMEM((BPW, D), jnp.float32),
            pltpu.VMEM((NCHUNK, SCHUNK), jnp.int32),
            pltpu.VMEM((SCHUNK, D), jnp.float32),
            pltpu.VMEM((SCHUNK, D), jnp.float32),
            pltpu.SemaphoreType.DMA,
            pltpu.SemaphoreType.DMA,
        ],
        compiler_params=pltpu.CompilerParams(use_tc_tiling_on_sc=False),
    )
    def k(user_tab, item_tab, uref, iref, sref, u_out, i_out, s_out,
          idx_v, rows_v, sidx_v, srows_a, srows_b, sem_a, sem_b):
        cid = lax.axis_index("c")
        sid = lax.axis_index("s")
        wid = sid * NC + cid
        base = wid * BPW
        pltpu.sync_copy(uref.at[pl.ds(base, BPW)], idx_v)
        pltpu.async_copy(user_tab.at[idx_v], rows_v, sem_a).wait()
        pltpu.sync_copy(rows_v, u_out.at[pl.ds(base, BPW)])
        pltpu.sync_copy(iref.at[pl.ds(base, BPW)], idx_v)
        pltpu.async_copy(item_tab.at[idx_v], rows_v, sem_a).wait()
        pltpu.sync_copy(rows_v, i_out.at[pl.ds(base, BPW)])
        pltpu.sync_copy(sref.at[wid], sidx_v)
        sbase = wid * SPW
        bufs = (srows_a, srows_b)
        sems = (sem_a, sem_b)
        copies = [
            pltpu.async_copy(item_tab.at[sidx_v.at[c]], bufs[c % 2],
                             sems[c % 2])
            for c in range(2)
        ]
        for c in range(NCHUNK):
            copies[c % 2].wait()
            pltpu.sync_copy(bufs[c % 2], s_out.at[pl.ds(sbase + c * SCHUNK,
                                                        SCHUNK)])
            if c + 2 < NCHUNK:
                copies[c % 2] = pltpu.async_copy(
                    item_tab.at[sidx_v.at[c + 2]], bufs[c % 2], sems[c % 2])

    return k(user_table, item_table, uidx, iidx, sidx3)


def _dot(a, b):
    return lax.dot_general(a, b, (((1,), (0,)), ((), ())),
                           preferred_element_type=jnp.float32,
                           precision=lax.Precision.HIGHEST)


def _tc_body(urows, irows, seq3, maskf, g_i, a_i, c_i, cl_i, num, text,
             g_tab, a_tab, c_tab, cl_tab, pos,
             W_num, b_num, g_num, be_num,
             W_text, b_text, g_text, be_text,
             W_seq, b_seq, W_a1, b_a1, W_a2, b_a2,
             W_u1, b_u1, W_u2, b_u2, W_i1, b_i1, W_i2, b_i2,
             out_ref):
    relu = lambda x: jnp.maximum(x, 0.0)

    def onehot_embed(idx_ref, tab_ref, n):
        iot = lax.broadcasted_iota(jnp.int32, (BB, n), 1)
        oh = (iot == idx_ref[...]).astype(jnp.float32)
        return _dot(oh, tab_ref[...])

    g_e = onehot_embed(g_i, g_tab, 3)
    a_e = onehot_embed(a_i, a_tab, 10)
    c_e = onehot_embed(c_i, c_tab, 1000)
    cl_e = onehot_embed(cl_i, cl_tab, 100)

    num_proj = relu((_dot(num[...], W_num[...]) + b_num[...]) * g_num[...]
                    + be_num[...])
    text_proj = relu((_dot(text[...], W_text[...]) + b_text[...])
                     * g_text[...] + be_text[...])

    wa1 = W_a1[...]
    wa2 = W_a2[...]
    ba1 = b_a1[...]
    ba2 = b_a2[...]
    acc_sum = jnp.zeros((BB, D), jnp.float32)
    acc_max = jnp.full((BB, D), -jnp.inf, jnp.float32)
    m = jnp.full((BB, 1), -jnp.inf, jnp.float32)
    l = jnp.zeros((BB, 1), jnp.float32)
    acc_att = jnp.zeros((BB, D), jnp.float32)
    for s in range(SEQ):
        mc = maskf[:, s:s + 1]
        xs = (seq3[s] + pos[s:s + 1, :]) * mc
        acc_sum = acc_sum + xs
        acc_max = jnp.maximum(acc_max, xs)
        h = relu(_dot(xs, wa1) + ba1)
        lg = (_dot(h, wa2) + ba2) * mc - 1e9 * (1.0 - mc)
        nm = jnp.maximum(m, lg)
        sc = jnp.exp(m - nm)
        p = jnp.exp(lg - nm)
        l = l * sc + p
        acc_att = acc_att * sc + p * xs
        m = nm
    valid = jnp.sum(maskf[...], axis=1, keepdims=True)
    mean_p = acc_sum / (valid + 1e-8)
    att_p = acc_att / l

    ws = W_seq[...]
    seq_embed = relu(_dot(mean_p, ws[0:D]) + _dot(acc_max, ws[D:2 * D])
                     + _dot(att_p, ws[2 * D:3 * D]) + b_seq[...])

    wu1 = W_u1[...]
    u = relu(_dot(urows[...], wu1[0:D]) + _dot(g_e, wu1[D:2 * D])
             + _dot(a_e, wu1[2 * D:3 * D]) + _dot(c_e, wu1[3 * D:4 * D])
             + _dot(cl_e, wu1[4 * D:5 * D]) + _dot(num_proj, wu1[5 * D:6 * D])
             + _dot(seq_embed, wu1[6 * D:7 * D]) + b_u1[...])
    u = relu(_dot(u, W_u2[...]) + b_u2[...])

    wi1 = W_i1[...]
    iv = relu(_dot(irows[...], wi1[0:D]) + _dot(text_proj, wi1[D:2 * D])
              + b_i1[...])
    iv = relu(_dot(iv, W_i2[...]) + b_i2[...])

    un = jnp.maximum(jnp.sqrt(jnp.sum(u * u, axis=1, keepdims=True)), 1e-12)
    inn = jnp.maximum(jnp.sqrt(jnp.sum(iv * iv, axis=1, keepdims=True)),
                      1e-12)
    out_ref[...] = jnp.sum(u * iv, axis=1, keepdims=True) / (un * inn)


def _row_spec(cols):
    return pl.BlockSpec((BB, cols), lambda i: (i, 0))


def _full_spec(shape):
    nd = len(shape)
    return pl.BlockSpec(shape, lambda i: (0,) * nd)


def _tc_kwargs():
    in_specs = [
        _row_spec(D),
        _row_spec(D),
        pl.BlockSpec((SEQ, BB, D), lambda i: (0, i, 0)),
        _row_spec(SEQ),
        _row_spec(1), _row_spec(1), _row_spec(1), _row_spec(1),
        _row_spec(16),
        _row_spec(128),
        _full_spec((3, D)), _full_spec((10, D)), _full_spec((1000, D)),
        _full_spec((100, D)), _full_spec((SEQ, D)),
        _full_spec((16, D)), _full_spec((1, D)), _full_spec((1, D)),
        _full_spec((1, D)),
        _full_spec((128, D)), _full_spec((1, D)), _full_spec((1, D)),
        _full_spec((1, D)),
        _full_spec((3 * D, D)), _full_spec((1, D)),
        _full_spec((D, D // 2)), _full_spec((1, D // 2)),
        _full_spec((D // 2, 1)), _full_spec((1, 1)),
        _full_spec((7 * D, 128)), _full_spec((1, 128)),
        _full_spec((128, D)), _full_spec((1, D)),
        _full_spec((2 * D, 128)), _full_spec((1, 128)),
        _full_spec((128, D)), _full_spec((1, D)),
    ]
    return dict(
        grid=(NB,),
        in_specs=in_specs,
        out_specs=pl.BlockSpec((BB, 1), lambda i: (i, 0)),
        out_shape=jax.ShapeDtypeStruct((B, 1), jnp.float32),
        compiler_params=pltpu.CompilerParams(
            dimension_semantics=("arbitrary",)),
    )


def _dense_args(u_rows, i_rows, seq3, sequence_mask, gender, age_range, city,
                cluster_id, user_numeric, item_text_feat, gender_table,
                age_table, city_table, cluster_table, position_table,
                W_num, b_num, g_num, be_num, W_text, b_text, g_text, be_text,
                W_seq, b_seq, W_a1, b_a1, W_a2, b_a2, W_u1, b_u1, W_u2, b_u2,
                W_i1, b_i1, W_i2, b_i2):
    r1 = lambda v: v.reshape(1, -1)
    k = 1.0 / jnp.sqrt(jnp.float32(1.0 + 1e-5))
    col = lambda v: v.astype(jnp.int32).reshape(B, 1)
    return (
        u_rows, i_rows, seq3, sequence_mask.astype(jnp.float32),
        col(gender), col(age_range), col(city), col(cluster_id),
        user_numeric, item_text_feat,
        gender_table, age_table, city_table, cluster_table, position_table,
        W_num, r1(b_num), r1(g_num) * k, r1(be_num),
        W_text, r1(b_text), r1(g_text) * k, r1(be_text),
        W_seq, r1(b_seq), W_a1, r1(b_a1), W_a2, r1(b_a2),
        W_u1, r1(b_u1), W_u2, r1(b_u2), W_i1, r1(b_i1), W_i2, r1(b_i2),
    )


def kernel(user_id, item_id, gender, age_range, city, cluster_id,
           user_numeric, item_text_feat, user_sequence, sequence_mask,
           user_table, gender_table, age_table, city_table, cluster_table,
           item_table, position_table, W_num, b_num, g_num, be_num,
           W_text, b_text, g_text, be_text, W_seq, b_seq, W_a1, b_a1,
           W_a2, b_a2, W_u1, b_u1, W_u2, b_u2, W_i1, b_i1, W_i2, b_i2):
    uidx = user_id.astype(jnp.int32)
    iidx = item_id.astype(jnp.int32)
    sidx3 = user_sequence.astype(jnp.int32).T.reshape(NW, NCHUNK, SCHUNK)
    u_rows, i_rows, s_rows = _sc_gather(user_table, item_table, uidx, iidx,
                                        sidx3)
    seq3 = s_rows.reshape(SEQ, B, D)
    args = _dense_args(u_rows, i_rows, seq3, sequence_mask, gender, age_range,
                       city, cluster_id, user_numeric, item_text_feat,
                       gender_table, age_table, city_table, cluster_table,
                       position_table, W_num, b_num, g_num, be_num, W_text,
                       b_text, g_text, be_text, W_seq, b_seq, W_a1, b_a1,
                       W_a2, b_a2, W_u1, b_u1, W_u2, b_u2, W_i1, b_i1,
                       W_i2, b_i2)
    out = pl.pallas_call(_tc_body, **_tc_kwargs())(*args)
    return out.reshape(B)
